# Initial kernel scaffold; baseline (speedup 1.0000x reference)
import jax
import jax.numpy as jnp
from jax.experimental import pallas as pl


def kernel(H, xyz, edge_index, edge_struct, edge_rest_lengths, We1, be1, We2, be2, Wg1, bg1, Wg2, bg2, Wn1, bn1, Wn2, bn2, ln_g, ln_b):
    raise NotImplementedError("write your pallas kernel here")


# speedup vs baseline: 2.5127x; 2.5127x over previous
import functools

import jax
import jax.numpy as jnp
from jax import lax
from jax.experimental import pallas as pl
from jax.experimental.pallas import tpu as pltpu
from jax.experimental.pallas import tpu_sc as plsc

_HIGH = lax.Precision.HIGHEST

_NC = 2
_NS = 16
_NW = _NC * _NS
_GR = 80
_CH = 5


def _silu(x):
    return x * jax.nn.sigmoid(x)


def _dot(a, b):
    return lax.dot_general(a, b, (((1,), (0,)), ((), ())), precision=_HIGH)


def _prep_body(h_ref, x_ref, wa_ref, wb_ref, ti_ref, tj_ref):
    h = h_ref[...]
    x = x_ref[...]
    ti_ref[...] = jnp.concatenate([_dot(h, wa_ref[...]), x], axis=1)
    tj_ref[...] = jnp.concatenate([_dot(h, wb_ref[...]), -x], axis=1)


def _edge_body(gi_ref, gj_ref, st_ref, rest_ref, w1sc_ref, w1st_ref, be1_ref,
               we2_ref, be2_ref, wg1sc_ref, wg1st_ref, bg1_ref, wg2_ref,
               bg2_ref, out_ref):
    s = gi_ref[...] + gj_ref[...]
    sl = s[:, 0:64]
    dvec = s[:, 64:80]
    dist2 = jnp.sum(dvec * dvec, axis=1, keepdims=True)
    dist = jnp.sqrt(dist2 + 1e-9)
    rest = rest_ref[...]
    delta = (dist - rest) / (rest + 1e-9)
    st = st_ref[...]
    w1sc = w1sc_ref[...]
    h1 = (sl + dist2 * w1sc[0:1, :] + delta * w1sc[1:2, :]
          + _dot(st, w1st_ref[...]) + be1_ref[...])
    e = _silu(_dot(_silu(h1), we2_ref[...]) + be2_ref[...])
    wg1sc = wg1sc_ref[...]
    g1 = (dist2 * wg1sc[0:1, :] + delta * wg1sc[1:2, :]
          + _dot(st, wg1st_ref[...]) + bg1_ref[...])
    g2 = jnp.sum(_silu(g1) * wg2_ref[...], axis=1, keepdims=True) + bg2_ref[...]
    out_ref[...] = e * jax.nn.sigmoid(g2)


def _node_body(h_ref, a0_ref, a1_ref, wnh_ref, wna_ref, bn1_ref, wn2_ref,
               bn2_ref, lg_ref, lb_ref, out_ref):
    h = h_ref[...]
    agg = a0_ref[...] + a1_ref[...]
    t = _silu(_dot(h, wnh_ref[...]) + _dot(agg, wna_ref[...]) + bn1_ref[...])
    res = h + _dot(t, wn2_ref[...]) + bn2_ref[...]
    mu = jnp.mean(res, axis=1, keepdims=True)
    c = res - mu
    var = jnp.mean(c * c, axis=1, keepdims=True)
    out_ref[...] = c * lax.rsqrt(var + 1e-5) * lg_ref[...] + lb_ref[...]


def _make_gather(E, N, W):
    rows = E // _GR
    rt = rows // _NW
    nmac = rt // _CH
    epm = _GR * _CH
    ept = E // _NW
    assert rt * _NW == rows and nmac * _CH == rt
    mesh = plsc.VectorSubcoreMesh(core_axis_name="c", subcore_axis_name="s",
                                  num_cores=_NC, num_subcores=_NS)
    f32 = jnp.float32

    @functools.partial(
        pl.kernel, mesh=mesh,
        out_type=(jax.ShapeDtypeStruct((E, W), f32),
                  jax.ShapeDtypeStruct((E, W), f32)),
        scratch_types=[
            pltpu.VMEM((rt, _GR), jnp.int32),
            pltpu.VMEM((rt, _GR), jnp.int32),
            pltpu.VMEM((epm, W), f32),
            pltpu.VMEM((epm, W), f32),
            pltpu.SemaphoreType.DMA,
            pltpu.SemaphoreType.DMA,
        ],
        compiler_params=pltpu.CompilerParams(use_tc_tiling_on_sc=False),
    )
    def gather_k(ti_hbm, tj_hbm, ii_hbm, jj_hbm, gi_hbm, gj_hbm,
                 ii_v, jj_v, bufi, bufj, semg, semo):
        wid = lax.axis_index("s") * _NC + lax.axis_index("c")
        ebase = wid * ept
        pltpu.sync_copy(ii_hbm.at[wid], ii_v)
        pltpu.sync_copy(jj_hbm.at[wid], jj_v)

        @pl.loop(0, nmac)
        def _mac(m):
            @pl.when(m > 0)
            def _():
                pltpu.make_async_copy(bufi, gi_hbm.at[pl.ds(0, epm)], semo).wait()
                pltpu.make_async_copy(bufj, gj_hbm.at[pl.ds(0, epm)], semo).wait()
            descs = []
            for k in range(_CH):
                descs.append(pltpu.async_copy(
                    ti_hbm.at[ii_v.at[m * _CH + k]],
                    bufi.at[pl.ds(k * _GR, _GR)], semg))
                descs.append(pltpu.async_copy(
                    tj_hbm.at[jj_v.at[m * _CH + k]],
                    bufj.at[pl.ds(k * _GR, _GR)], semg))
            for dsc in descs:
                dsc.wait()
            pltpu.async_copy(bufi, gi_hbm.at[pl.ds(ebase + m * epm, epm)], semo)
            pltpu.async_copy(bufj, gj_hbm.at[pl.ds(ebase + m * epm, epm)], semo)

        pltpu.make_async_copy(bufi, gi_hbm.at[pl.ds(0, epm)], semo).wait()
        pltpu.make_async_copy(bufj, gj_hbm.at[pl.ds(0, epm)], semo).wait()

    return gather_k


def _make_scatter(E, N, D):
    rows = E // _GR
    rt = rows // _NW
    nmac = rt // _CH
    epm = _GR * _CH
    ept = E // _NW
    rps = N // _NS
    assert rps * _NS == N
    mesh = plsc.VectorSubcoreMesh(core_axis_name="c", subcore_axis_name="s",
                                  num_cores=_NC, num_subcores=_NS)
    f32 = jnp.float32

    @functools.partial(
        pl.kernel, mesh=mesh,
        out_type=jax.ShapeDtypeStruct((_NC, N, D), f32),
        scratch_types=[
            pltpu.VMEM((rt, _GR), jnp.int32),
            pltpu.VMEM((epm, D), f32),
            pltpu.VMEM_SHARED((N, D), f32),
            pltpu.SemaphoreType.DMA,
        ],
        compiler_params=pltpu.CompilerParams(use_tc_tiling_on_sc=False),
    )
    def scatter_k(msg_hbm, ii_hbm, zero_hbm, aggp_hbm, ii_v, mbuf, agg_sh, sem):
        cid = lax.axis_index("c")
        sid = lax.axis_index("s")
        wid = sid * _NC + cid
        @pl.when(sid == 0)
        def _():
            pltpu.sync_copy(zero_hbm, agg_sh)
        pltpu.sync_copy(ii_hbm.at[wid], ii_v)
        plsc.subcore_barrier()

        @pl.loop(0, nmac)
        def _mac(m):
            pltpu.sync_copy(msg_hbm.at[pl.ds(wid * ept + m * epm, epm)], mbuf)
            for k in range(_CH):
                pltpu.sync_copy(mbuf.at[pl.ds(k * _GR, _GR)],
                                agg_sh.at[ii_v.at[m * _CH + k]], add=True)

        plsc.subcore_barrier()

        @pl.when(sid == 0)
        def _():
            pltpu.sync_copy(agg_sh, aggp_hbm.at[cid])

    return scatter_k


def kernel(H, xyz, edge_index, edge_struct, edge_rest_lengths,
           We1, be1, We2, be2, Wg1, bg1, Wg2, bg2,
           Wn1, bn1, Wn2, bn2, ln_g, ln_b):
    B, N, d = H.shape
    E = edge_index.shape[1]
    d_edge = We2.shape[1]
    W = 80
    assert E % (_NW * _GR * _CH) == 0 and N % _NS == 0
    f32 = jnp.float32

    H2 = H[0]
    xyzp = jnp.pad(xyz[0], ((0, 0), (0, 13)))
    rt = E // _GR // _NW
    ii = edge_index[0].reshape(_NW, rt, _GR)
    jj = edge_index[1].reshape(_NW, rt, _GR)

    BN = 1000
    ti, tj = pl.pallas_call(
        _prep_body,
        grid=(N // BN,),
        in_specs=[
            pl.BlockSpec((BN, d), lambda i: (i, 0)),
            pl.BlockSpec((BN, 16), lambda i: (i, 0)),
            pl.BlockSpec((d, d_edge), lambda i: (0, 0)),
            pl.BlockSpec((d, d_edge), lambda i: (0, 0)),
        ],
        out_specs=[
            pl.BlockSpec((BN, W), lambda i: (i, 0)),
            pl.BlockSpec((BN, W), lambda i: (i, 0)),
        ],
        out_shape=[
            jax.ShapeDtypeStruct((N, W), f32),
            jax.ShapeDtypeStruct((N, W), f32),
        ],
    )(H2, xyzp, We1[:d], We1[d:2 * d])

    gi, gj = _make_gather(E, N, W)(ti, tj, ii, jj)

    BE = 2000
    e_msg = pl.pallas_call(
        _edge_body,
        grid=(E // BE,),
        in_specs=[
            pl.BlockSpec((BE, W), lambda i: (i, 0)),
            pl.BlockSpec((BE, W), lambda i: (i, 0)),
            pl.BlockSpec((BE, 8), lambda i: (i, 0)),
            pl.BlockSpec((BE, 1), lambda i: (i, 0)),
            pl.BlockSpec((2, d_edge), lambda i: (0, 0)),
            pl.BlockSpec((8, d_edge), lambda i: (0, 0)),
            pl.BlockSpec((1, d_edge), lambda i: (0, 0)),
            pl.BlockSpec((d_edge, d_edge), lambda i: (0, 0)),
            pl.BlockSpec((1, d_edge), lambda i: (0, 0)),
            pl.BlockSpec((2, 32), lambda i: (0, 0)),
            pl.BlockSpec((8, 32), lambda i: (0, 0)),
            pl.BlockSpec((1, 32), lambda i: (0, 0)),
            pl.BlockSpec((1, 32), lambda i: (0, 0)),
            pl.BlockSpec((1, 1), lambda i: (0, 0)),
        ],
        out_specs=pl.BlockSpec((BE, d_edge), lambda i: (i, 0)),
        out_shape=jax.ShapeDtypeStruct((E, d_edge), f32),
    )(gi, gj, edge_struct, edge_rest_lengths.reshape(E, 1),
      We1[2 * d:2 * d + 2], We1[2 * d + 2:2 * d + 10], be1.reshape(1, -1),
      We2, be2.reshape(1, -1),
      Wg1[0:2], Wg1[2:10], bg1.reshape(1, -1), Wg2.reshape(1, -1),
      bg2.reshape(1, 1))

    aggp = _make_scatter(E, N, d_edge)(e_msg, ii, jnp.zeros((N, d_edge), f32))

    dh = Wn1.shape[1]
    out = pl.pallas_call(
        _node_body,
        grid=(N // BN,),
        in_specs=[
            pl.BlockSpec((BN, d), lambda i: (i, 0)),
            pl.BlockSpec((BN, d_edge), lambda i: (i, 0)),
            pl.BlockSpec((BN, d_edge), lambda i: (i, 0)),
            pl.BlockSpec((d, dh), lambda i: (0, 0)),
            pl.BlockSpec((d_edge, dh), lambda i: (0, 0)),
            pl.BlockSpec((1, dh), lambda i: (0, 0)),
            pl.BlockSpec((dh, d), lambda i: (0, 0)),
            pl.BlockSpec((1, d), lambda i: (0, 0)),
            pl.BlockSpec((1, d), lambda i: (0, 0)),
            pl.BlockSpec((1, d), lambda i: (0, 0)),
        ],
        out_specs=pl.BlockSpec((BN, d), lambda i: (i, 0)),
        out_shape=jax.ShapeDtypeStruct((N, d), f32),
    )(H2, aggp[0], aggp[1], Wn1[:d], Wn1[d:], bn1.reshape(1, -1),
      Wn2, bn2.reshape(1, -1), ln_g.reshape(1, -1), ln_b.reshape(1, -1))

    return out.reshape(B, N, d)
